# baseline (speedup 1.0000x reference)
import functools

import jax
import jax.numpy as jnp
from jax import lax
from jax.experimental import pallas as pl
from jax.experimental.pallas import tpu as pltpu
from jax.experimental.pallas import tpu_sc as plsc

B = 16384
D = 512
V = 10000
L = 16
NC = 2
NS = 16
NW = NC * NS

B_TC = 6144
B_SC = B - B_TC
BPW = B_SC // NW
CH = 32
NCH = BPW // CH
NACC = 4

R = 512
G = B_TC // R

_mesh = plsc.VectorSubcoreMesh(core_axis_name="c", subcore_axis_name="s")


@functools.partial(
    pl.kernel,
    mesh=_mesh,
    out_type=jax.ShapeDtypeStruct((1, NW * L), jnp.float32),
    scratch_types=[
        pltpu.VMEM((BPW,), jnp.int32),
        pltpu.VMEM((CH, D), jnp.float32),
        pltpu.VMEM((CH, D), jnp.float32),
        pltpu.VMEM((CH, D), jnp.float32),
        pltpu.VMEM((CH, D), jnp.float32),
        pltpu.VMEM((L,), jnp.float32),
        pltpu.SemaphoreType.DMA,
        pltpu.SemaphoreType.DMA,
        pltpu.SemaphoreType.DMA,
        pltpu.SemaphoreType.DMA,
    ],
)
def _sc_partials(features_hbm, target_hbm, center_hbm, out_hbm,
                 idx_v, fbuf0, fbuf1, cbuf0, cbuf1, accv,
                 semf0, semf1, semc0, semc1):
    wid = lax.axis_index("s") * NC + lax.axis_index("c")
    base = wid * BPW
    pltpu.sync_copy(target_hbm.at[pl.ds(base, BPW)], idx_v)

    fbufs = (fbuf0, fbuf1)
    cbufs = (cbuf0, cbuf1)
    semfs = (semf0, semf1)
    semcs = (semc0, semc1)

    def start(g):
        b = g % 2
        cpf = pltpu.async_copy(
            features_hbm.at[pl.ds(base + g * CH, CH)], fbufs[b], semfs[b])
        cpc = pltpu.async_copy(
            center_hbm.at[idx_v.at[pl.ds(g * CH, CH)]], cbufs[b], semcs[b])
        return cpf, cpc

    def compute(b, accs):
        fbuf = fbufs[b]
        cbuf = cbufs[b]

        def row_body(r, accs):
            accs = list(accs)
            for v in range(D // L):
                df = fbuf[r, pl.ds(v * L, L)] - cbuf[r, pl.ds(v * L, L)]
                accs[v % NACC] = accs[v % NACC] + df * df
            return tuple(accs)

        return lax.fori_loop(0, CH, row_body, tuple(accs))

    accs = tuple(jnp.zeros((L,), jnp.float32) for _ in range(NACC))
    pending = start(0)
    for g in range(NCH):
        cur = pending
        if g + 1 < NCH:
            pending = start(g + 1)
        cur[0].wait()
        cur[1].wait()
        accs = compute(g % 2, accs)

    accs = list(accs)
    total = accs[0]
    for a in accs[1:]:
        total = total + a
    accv[...] = total
    pltpu.sync_copy(accv, out_hbm.at[0, pl.ds(wid * L, L)])


def _tc_body(tgt_ref, f_ref, c_ref, o_ref, gath_ref):
    i = pl.program_id(0)

    @pl.when(i == 0)
    def _():
        o_ref[...] = jnp.zeros_like(o_ref)

    def cp(r16, _):
        for j in range(16):
            r = r16 * 16 + j
            idx = tgt_ref[B_SC + i * R + r]
            gath_ref[pl.ds(r, 1), :] = c_ref[pl.ds(idx, 1), :]
        return 0

    lax.fori_loop(0, R // 16, cp, 0)
    d = f_ref[...] - gath_ref[...]
    o_ref[...] += jnp.sum(d * d, axis=0, keepdims=True)


_tc_partial = pl.pallas_call(
    _tc_body,
    grid_spec=pltpu.PrefetchScalarGridSpec(
        num_scalar_prefetch=1,
        grid=(G,),
        in_specs=[
            pl.BlockSpec((R, D), lambda i, tgt: (B_SC // R + i, 0)),
            pl.BlockSpec((V, D), lambda i, tgt: (0, 0)),
        ],
        out_specs=pl.BlockSpec((1, D), lambda i, tgt: (0, 0)),
        scratch_shapes=[pltpu.VMEM((R, D), jnp.float32)],
    ),
    out_shape=jax.ShapeDtypeStruct((1, D), jnp.float32),
)


def kernel(features, target, center):
    tgt = target.astype(jnp.int32)
    psc = _sc_partials(features, tgt, center)
    ptc = _tc_partial(tgt, features, center)
    return jnp.sum(psc + ptc) * (1.0 / B)


# speedup vs baseline: 1.0450x; 1.0450x over previous
import functools

import jax
import jax.numpy as jnp
from jax import lax
from jax.experimental import pallas as pl
from jax.experimental.pallas import tpu as pltpu
from jax.experimental.pallas import tpu_sc as plsc

B = 16384
D = 512
V = 10000
L = 16
NC = 2
NS = 16
NW = NC * NS

B_TC = 5632
B_SC = B - B_TC
BPW = B_SC // NW
CH = 48
NCH = BPW // CH
NACC = 4

R = 512
G = B_TC // R

_mesh = plsc.VectorSubcoreMesh(core_axis_name="c", subcore_axis_name="s")


@functools.partial(
    pl.kernel,
    mesh=_mesh,
    out_type=jax.ShapeDtypeStruct((1, NW * L), jnp.float32),
    scratch_types=[
        pltpu.VMEM((BPW,), jnp.int32),
        pltpu.VMEM((CH, D), jnp.float32),
        pltpu.VMEM((CH, D), jnp.float32),
        pltpu.VMEM((CH, D), jnp.float32),
        pltpu.VMEM((CH, D), jnp.float32),
        pltpu.VMEM((L,), jnp.float32),
        pltpu.SemaphoreType.DMA,
        pltpu.SemaphoreType.DMA,
        pltpu.SemaphoreType.DMA,
        pltpu.SemaphoreType.DMA,
    ],
)
def _sc_partials(features_hbm, target_hbm, center_hbm, out_hbm,
                 idx_v, fbuf0, fbuf1, cbuf0, cbuf1, accv,
                 semf0, semf1, semc0, semc1):
    wid = lax.axis_index("s") * NC + lax.axis_index("c")
    base = wid * BPW
    pltpu.sync_copy(target_hbm.at[pl.ds(base, BPW)], idx_v)

    fbufs = (fbuf0, fbuf1)
    cbufs = (cbuf0, cbuf1)
    semfs = (semf0, semf1)
    semcs = (semc0, semc1)

    def start(g):
        b = g % 2
        cpf = pltpu.async_copy(
            features_hbm.at[pl.ds(base + g * CH, CH)], fbufs[b], semfs[b])
        cpc = pltpu.async_copy(
            center_hbm.at[idx_v.at[pl.ds(g * CH, CH)]], cbufs[b], semcs[b])
        return cpf, cpc

    def compute(b, accs):
        fbuf = fbufs[b]
        cbuf = cbufs[b]

        def row_body(r, accs):
            accs = list(accs)
            for v in range(D // L):
                df = fbuf[r, pl.ds(v * L, L)] - cbuf[r, pl.ds(v * L, L)]
                accs[v % NACC] = accs[v % NACC] + df * df
            return tuple(accs)

        return lax.fori_loop(0, CH, row_body, tuple(accs))

    accs = tuple(jnp.zeros((L,), jnp.float32) for _ in range(NACC))
    pending = start(0)
    for g in range(NCH):
        cur = pending
        if g + 1 < NCH:
            pending = start(g + 1)
        cur[0].wait()
        cur[1].wait()
        accs = compute(g % 2, accs)

    accs = list(accs)
    total = accs[0]
    for a in accs[1:]:
        total = total + a
    accv[...] = total
    pltpu.sync_copy(accv, out_hbm.at[0, pl.ds(wid * L, L)])


def _tc_body(tgt_ref, f_ref, c_ref, o_ref, gath_ref):
    i = pl.program_id(0)

    @pl.when(i == 0)
    def _():
        o_ref[...] = jnp.zeros_like(o_ref)

    def cp(r16, _):
        for j in range(16):
            r = r16 * 16 + j
            idx = tgt_ref[B_SC + i * R + r]
            gath_ref[pl.ds(r, 1), :] = c_ref[pl.ds(idx, 1), :]
        return 0

    lax.fori_loop(0, R // 16, cp, 0)
    d = f_ref[...] - gath_ref[...]
    o_ref[...] += jnp.sum(d * d, axis=0, keepdims=True)


_tc_partial = pl.pallas_call(
    _tc_body,
    grid_spec=pltpu.PrefetchScalarGridSpec(
        num_scalar_prefetch=1,
        grid=(G,),
        in_specs=[
            pl.BlockSpec((R, D), lambda i, tgt: (B_SC // R + i, 0)),
            pl.BlockSpec((V, D), lambda i, tgt: (0, 0)),
        ],
        out_specs=pl.BlockSpec((1, D), lambda i, tgt: (0, 0)),
        scratch_shapes=[pltpu.VMEM((R, D), jnp.float32)],
    ),
    out_shape=jax.ShapeDtypeStruct((1, D), jnp.float32),
)


def kernel(features, target, center):
    tgt = target.astype(jnp.int32)
    psc = _sc_partials(features, tgt, center)
    ptc = _tc_partial(tgt, features, center)
    return jnp.sum(psc + ptc) * (1.0 / B)


# speedup vs baseline: 1.0530x; 1.0077x over previous
import functools

import jax
import jax.numpy as jnp
from jax import lax
from jax.experimental import pallas as pl
from jax.experimental.pallas import tpu as pltpu
from jax.experimental.pallas import tpu_sc as plsc

B = 16384
D = 512
V = 10000
L = 16
NC = 2
NS = 16
NW = NC * NS

B_TC = 5632
B_SC = B - B_TC
BPW = B_SC // NW
CH = 48
NCH = BPW // CH
NACC = 4

R = 512
G = B_TC // R

_mesh = plsc.VectorSubcoreMesh(core_axis_name="c", subcore_axis_name="s")


@functools.partial(
    pl.kernel,
    mesh=_mesh,
    out_type=jax.ShapeDtypeStruct((1, NW * L), jnp.float32),
    scratch_types=[
        pltpu.VMEM((BPW,), jnp.int32),
        pltpu.VMEM((CH, D), jnp.float32),
        pltpu.VMEM((CH, D), jnp.float32),
        pltpu.VMEM((CH, D), jnp.float32),
        pltpu.VMEM((CH, D), jnp.float32),
        pltpu.VMEM((L,), jnp.float32),
        pltpu.SemaphoreType.DMA,
        pltpu.SemaphoreType.DMA,
        pltpu.SemaphoreType.DMA,
        pltpu.SemaphoreType.DMA,
    ],
)
def _sc_partials(features_hbm, target_hbm, center_hbm, out_hbm,
                 idx_v, fbuf0, fbuf1, cbuf0, cbuf1, accv,
                 semf0, semf1, semc0, semc1):
    wid = lax.axis_index("s") * NC + lax.axis_index("c")
    base = wid * BPW
    pltpu.sync_copy(target_hbm.at[pl.ds(base, BPW)], idx_v)

    fbufs = (fbuf0, fbuf1)
    cbufs = (cbuf0, cbuf1)
    semfs = (semf0, semf1)
    semcs = (semc0, semc1)

    def start(g):
        b = g % 2
        cpf = pltpu.async_copy(
            features_hbm.at[pl.ds(base + g * CH, CH)], fbufs[b], semfs[b])
        cpc = pltpu.async_copy(
            center_hbm.at[idx_v.at[pl.ds(g * CH, CH)]], cbufs[b], semcs[b])
        return cpf, cpc

    def compute(b, accs):
        fbuf = fbufs[b]
        cbuf = cbufs[b]

        def row_body(r, accs):
            accs = list(accs)
            for v in range(D // L):
                df = fbuf[r, pl.ds(v * L, L)] - cbuf[r, pl.ds(v * L, L)]
                accs[v % NACC] = accs[v % NACC] + df * df
            return tuple(accs)

        return lax.fori_loop(0, CH, row_body, tuple(accs))

    accs = tuple(jnp.zeros((L,), jnp.float32) for _ in range(NACC))
    pending = start(0)
    for g in range(NCH):
        cur = pending
        if g + 1 < NCH:
            pending = start(g + 1)
        cur[0].wait()
        cur[1].wait()
        accs = compute(g % 2, accs)

    accs = list(accs)
    total = accs[0]
    for a in accs[1:]:
        total = total + a
    accv[...] = total
    pltpu.sync_copy(accv, out_hbm.at[0, pl.ds(wid * L, L)])


def _tc_body(tgt_ref, f_ref, c_ref, o_ref, gath_ref):
    i = pl.program_id(0)

    @pl.when(i == 0)
    def _():
        o_ref[...] = jnp.zeros_like(o_ref)

    def cp(r32, _):
        for j in range(32):
            r = r32 * 32 + j
            idx = tgt_ref[B_SC + i * R + r]
            gath_ref[pl.ds(r, 1), :] = c_ref[pl.ds(idx, 1), :]
        return 0

    lax.fori_loop(0, R // 32, cp, 0)
    d = f_ref[...] - gath_ref[...]
    o_ref[...] += jnp.sum(d * d, axis=0, keepdims=True)


_tc_partial = pl.pallas_call(
    _tc_body,
    grid_spec=pltpu.PrefetchScalarGridSpec(
        num_scalar_prefetch=1,
        grid=(G,),
        in_specs=[
            pl.BlockSpec((R, D), lambda i, tgt: (B_SC // R + i, 0)),
            pl.BlockSpec((V, D), lambda i, tgt: (0, 0)),
        ],
        out_specs=pl.BlockSpec((1, D), lambda i, tgt: (0, 0)),
        scratch_shapes=[pltpu.VMEM((R, D), jnp.float32)],
    ),
    out_shape=jax.ShapeDtypeStruct((1, D), jnp.float32),
)


def kernel(features, target, center):
    tgt = target.astype(jnp.int32)
    psc = _sc_partials(features, tgt, center)
    ptc = _tc_partial(tgt, features, center)
    return jnp.sum(psc + ptc) * (1.0 / B)
